# baseline (speedup 1.0000x reference)
import jax
import jax.numpy as jnp
from jax.experimental import pallas as pl
from jax.experimental.pallas import tpu as pltpu


def _geom(W):
    W_o = -(-W // 16) * 16
    W_im = -(-(W_o + 2) // 16) * 16
    return W_o, W_im


def _head_body(shapes, *refs):
    n = len(shapes)
    x_refs = refs[:n]
    (wa_ref, ba_ref, w1_ref, c1b_ref, g1_ref, be1_ref, m1_ref, v1_ref,
     w2_ref, c2b_ref, g2_ref, be2_ref, m2_ref, v2r_ref, wp_ref,
     bp_ref) = refs[n:n + 16]
    o_refs = refs[n + 16:n + 16 + 5 * n]
    h_refs = refs[n + 16 + 5 * n:]

    C = wa_ref.shape[2]
    F = wa_ref.shape[1]
    P = wp_ref.shape[1]
    eps = 1e-5

    @pl.when(pl.program_id(0) == 0)
    def _():
        for h in h_refs:
            h[...] = jnp.zeros(h.shape, jnp.bfloat16)

    idn = (jax.lax.broadcasted_iota(jnp.int32, (F, F), 0) ==
           jax.lax.broadcasted_iota(jnp.int32, (F, F), 1)).astype(jnp.bfloat16)

    for i, (H, W, W_o, W_im) in enumerate(shapes):
        x_ref = x_refs[i]
        ob, oo, os_, oa, od = o_refs[5 * i:5 * i + 5]
        h0 = h_refs[2 * i]
        h1 = h_refs[2 * i + 1]
        M = H * W_o

        s1 = g1_ref[i] * jax.lax.rsqrt(v1_ref[i] + eps)
        b1 = (c1b_ref[i] - m1_ref[i]) * s1 + be1_ref[i]
        s2 = g2_ref[i] * jax.lax.rsqrt(v2r_ref[i] + eps)
        b2 = (c2b_ref[i] - m2_ref[i]) * s2 + be2_ref[i]

        if W_o != W:
            col = jax.lax.broadcasted_iota(jnp.int32, (M, F), 0) % W_o
            keep = col < W

        def to_im2col(v, dst):
            if W_o != W:
                v = jnp.where(keep, v, 0.0)
            img = v.astype(jnp.bfloat16).reshape(H, W_o, C)
            dst[1:H + 1, 1:W_o + 1, 0:C] = img
            dst[1:H + 1, 0:W_o, C:2 * C] = img
            dst[1:H + 1, 0:W_o - 1, 2 * C:3 * C] = img[:, 1:, :]

        aT = jnp.dot(wa_ref[i], x_ref[0].astype(jnp.bfloat16),
                     preferred_element_type=jnp.float32)
        aT = (aT + ba_ref[i]).astype(jnp.bfloat16)
        a = jax.lax.dot_general(aT, idn, (((0,), (0,)), ((), ())),
                                preferred_element_type=jnp.float32)
        to_im2col(a, h0)

        def conv3(src_ref, w_ref, s, b):
            acc = None
            for di in range(3):
                xs = src_ref[di:di + H, 0:W_o, :].reshape(M, 3 * C)
                d = jnp.dot(xs, w_ref[i, di],
                            preferred_element_type=jnp.float32)
                acc = d if acc is None else acc + d
            return jnp.maximum(acc * s + b, 0.0)

        to_im2col(conv3(h0, w1_ref, s1, b1), h1)
        v2 = conv3(h1, w2_ref, s2, b2)

        pT = jax.lax.dot_general(wp_ref[i], v2, (((1,), (1,)), ((), ())),
                                 preferred_element_type=jnp.float32)
        pT = pT + bp_ref[i]
        k = jax.lax.broadcasted_iota(jnp.int32, (P, M), 0) % 15
        pT = jnp.where(k == 4, jax.nn.sigmoid(pT),
                       jnp.where(k == 14, jax.nn.softplus(pT) + 1.0, pT))

        for aidx in range(3):
            base = 15 * aidx
            ob[0, 4 * aidx:4 * aidx + 4] = pT[base:base + 4]
            oo[0, aidx:aidx + 1] = pT[base + 4:base + 5]
            os_[0, 5 * aidx:5 * aidx + 5] = pT[base + 5:base + 10]
            oa[0, 4 * aidx:4 * aidx + 4] = pT[base + 10:base + 14]
            od[0, aidx:aidx + 1] = pT[base + 14:base + 15]


def kernel(feat0, feat1, feat2, feat3, adapt_w, adapt_b, c1_w, c1_b, bn1_g,
           bn1_b, bn1_m, bn1_v, c2_w, c2_b, bn2_g, bn2_b, bn2_m, bn2_v,
           pred_w, pred_b):
    L, F = adapt_b.shape
    C = feat0.shape[1]
    P = pred_b.shape[1]
    feats = [feat0, feat1, feat2, feat3]
    bf16 = jnp.bfloat16
    B = feat0.shape[0]

    wa = adapt_w.reshape(L, F, C).astype(bf16)
    w1 = c1_w.transpose(0, 3, 4, 2, 1).reshape(L, 3, 3 * C, F).astype(bf16)
    w2 = c2_w.transpose(0, 3, 4, 2, 1).reshape(L, 3, 3 * F, F).astype(bf16)
    wp = pred_w.reshape(L, P, F)

    vec = lambda a: a.reshape(L, 1, F)
    ba = adapt_b.reshape(L, F, 1)
    bp = pred_b.reshape(L, P, 1)

    shapes = []
    xs = []
    for f in feats:
        _, _, H, W = f.shape
        W_o, W_im = _geom(W)
        shapes.append((H, W, W_o, W_im))
        if W_o != W:
            f = jnp.pad(f, ((0, 0), (0, 0), (0, 0), (0, W_o - W)))
        xs.append(f.reshape(B, C, H * W_o))

    full = lambda a: pl.BlockSpec(a.shape, lambda b: (0,) * a.ndim)
    params = (wa, ba, w1, vec(c1_b), vec(bn1_g), vec(bn1_b), vec(bn1_m),
              vec(bn1_v), w2, vec(c2_b), vec(bn2_g), vec(bn2_b), vec(bn2_m),
              vec(bn2_v), wp, bp)
    in_specs = (
        [pl.BlockSpec((1, C, H * W_o), lambda b: (b, 0, 0))
         for (H, W, W_o, W_im) in shapes]
        + [full(a) for a in params]
    )
    out_specs = []
    out_shape = []
    for (H, W, W_o, W_im) in shapes:
        for ch in (12, 3, 15, 12, 3):
            out_specs.append(pl.BlockSpec((1, ch, H * W_o),
                                          lambda b: (b, 0, 0)))
            out_shape.append(
                jax.ShapeDtypeStruct((B, ch, H * W_o), jnp.float32))
    scratch_shapes = []
    for (H, W, W_o, W_im) in shapes:
        scratch_shapes += [pltpu.VMEM((H + 2, W_im, 3 * C), bf16)] * 2

    ps = pl.pallas_call(
        lambda *refs: _head_body(shapes, *refs),
        grid=(B,),
        in_specs=in_specs,
        out_specs=out_specs,
        out_shape=out_shape,
        scratch_shapes=scratch_shapes,
        compiler_params=pltpu.CompilerParams(
            dimension_semantics=("arbitrary",)),
    )(*xs, *params)

    outs = []
    for li, (H, W, W_o, W_im) in enumerate(shapes):
        leaves = []
        for j, ch in enumerate((12, 3, 15, 12, 3)):
            t = ps[5 * li + j].reshape(B, 3, ch // 3, H, W_o)
            if W_o != W:
                t = t[..., :W]
            leaves.append(t)
        outs.append(tuple(leaves))
    return tuple(outs)


# speedup vs baseline: 1.0487x; 1.0487x over previous
import jax
import jax.numpy as jnp
from jax.experimental import pallas as pl
from jax.experimental.pallas import tpu as pltpu


def _geom(W):
    W_o = -(-W // 16) * 16
    W_im = -(-(W_o + 2) // 16) * 16
    return W_o, W_im


def _head_body(shapes, *refs):
    n = len(shapes)
    x_refs = refs[:n]
    (wa_ref, ba_ref, w1_ref, c1b_ref, g1_ref, be1_ref, m1_ref, v1_ref,
     w2_ref, c2b_ref, g2_ref, be2_ref, m2_ref, v2r_ref, wp_ref,
     bp_ref) = refs[n:n + 16]
    o_refs = refs[n + 16:n + 16 + 5 * n]
    h_refs = refs[n + 16 + 5 * n:]

    C = wa_ref.shape[1]
    F = wa_ref.shape[2]
    P = wp_ref.shape[1]
    eps = 1e-5

    @pl.when(pl.program_id(0) == 0)
    def _():
        for h in h_refs:
            h[...] = jnp.zeros(h.shape, jnp.bfloat16)

    for i, (H, W, W_o, W_im) in enumerate(shapes):
        x_ref = x_refs[i]
        ob, oo, os_, oa, od = o_refs[5 * i:5 * i + 5]
        h0 = h_refs[2 * i]
        h1 = h_refs[2 * i + 1]
        M = H * W_o

        s1 = g1_ref[i] * jax.lax.rsqrt(v1_ref[i] + eps)
        b1 = (c1b_ref[i] - m1_ref[i]) * s1 + be1_ref[i]
        s2 = g2_ref[i] * jax.lax.rsqrt(v2r_ref[i] + eps)
        b2 = (c2b_ref[i] - m2_ref[i]) * s2 + be2_ref[i]

        if W_o != W:
            col = jax.lax.broadcasted_iota(jnp.int32, (M, F), 0) % W_o
            keep = col < W

        def to_im2col(v, dst):
            if W_o != W:
                v = jnp.where(keep, v, 0.0)
            img = v.astype(jnp.bfloat16).reshape(H, W_o, C)
            dst[1:H + 1, 1:W_o + 1, 0:C] = img
            dst[1:H + 1, 0:W_o, C:2 * C] = img
            dst[1:H + 1, 0:W_o - 1, 2 * C:3 * C] = img[:, 1:, :]

        x = x_ref[0].reshape(M, C)
        a = jnp.dot(x, wa_ref[i], preferred_element_type=jnp.float32)
        to_im2col(a + ba_ref[i], h0)

        def conv3(src_ref, w_ref, s, b):
            acc = None
            for di in range(3):
                xs = src_ref[di:di + H, 0:W_o, :].reshape(M, 3 * C)
                d = jnp.dot(xs, w_ref[i, di],
                            preferred_element_type=jnp.float32)
                acc = d if acc is None else acc + d
            return jnp.maximum(acc * s + b, 0.0)

        to_im2col(conv3(h0, w1_ref, s1, b1), h1)
        v2 = conv3(h1, w2_ref, s2, b2)

        pT = jax.lax.dot_general(wp_ref[i], v2, (((1,), (1,)), ((), ())),
                                 preferred_element_type=jnp.float32)
        pT = pT + bp_ref[i]
        k = jax.lax.broadcasted_iota(jnp.int32, (P, M), 0) % 15
        pT = jnp.where(k == 4, jax.nn.sigmoid(pT),
                       jnp.where(k == 14, jax.nn.softplus(pT) + 1.0, pT))

        for aidx in range(3):
            base = 15 * aidx
            ob[0, 4 * aidx:4 * aidx + 4] = pT[base:base + 4]
            oo[0, aidx:aidx + 1] = pT[base + 4:base + 5]
            os_[0, 5 * aidx:5 * aidx + 5] = pT[base + 5:base + 10]
            oa[0, 4 * aidx:4 * aidx + 4] = pT[base + 10:base + 14]
            od[0, aidx:aidx + 1] = pT[base + 14:base + 15]


def kernel(feat0, feat1, feat2, feat3, adapt_w, adapt_b, c1_w, c1_b, bn1_g,
           bn1_b, bn1_m, bn1_v, c2_w, c2_b, bn2_g, bn2_b, bn2_m, bn2_v,
           pred_w, pred_b):
    L, F = adapt_b.shape
    C = feat0.shape[1]
    P = pred_b.shape[1]
    feats = [feat0, feat1, feat2, feat3]
    bf16 = jnp.bfloat16
    B = feat0.shape[0]

    wa = adapt_w.reshape(L, F, C).transpose(0, 2, 1).astype(bf16)
    w1 = c1_w.transpose(0, 3, 4, 2, 1).reshape(L, 3, 3 * C, F).astype(bf16)
    w2 = c2_w.transpose(0, 3, 4, 2, 1).reshape(L, 3, 3 * F, F).astype(bf16)
    wp = pred_w.reshape(L, P, F)

    vec = lambda a: a.reshape(L, 1, F)
    ba = adapt_b.reshape(L, 1, F)
    bp = pred_b.reshape(L, P, 1)

    shapes = []
    xs = []
    for f in feats:
        _, _, H, W = f.shape
        W_o, W_im = _geom(W)
        shapes.append((H, W, W_o, W_im))
        x = f.transpose(0, 2, 3, 1)
        if W_o != W:
            x = jnp.pad(x, ((0, 0), (0, 0), (0, W_o - W), (0, 0)))
        xs.append(x.astype(bf16))

    full = lambda a: pl.BlockSpec(a.shape, lambda b: (0,) * a.ndim)
    params = (wa, ba, w1, vec(c1_b), vec(bn1_g), vec(bn1_b), vec(bn1_m),
              vec(bn1_v), w2, vec(c2_b), vec(bn2_g), vec(bn2_b), vec(bn2_m),
              vec(bn2_v), wp, bp)
    in_specs = (
        [pl.BlockSpec((1, H, W_o, C), lambda b: (b, 0, 0, 0))
         for (H, W, W_o, W_im) in shapes]
        + [full(a) for a in params]
    )
    out_specs = []
    out_shape = []
    for (H, W, W_o, W_im) in shapes:
        for ch in (12, 3, 15, 12, 3):
            out_specs.append(pl.BlockSpec((1, ch, H * W_o),
                                          lambda b: (b, 0, 0)))
            out_shape.append(
                jax.ShapeDtypeStruct((B, ch, H * W_o), jnp.float32))
    scratch_shapes = []
    for (H, W, W_o, W_im) in shapes:
        scratch_shapes += [pltpu.VMEM((H + 2, W_im, 3 * C), bf16)] * 2

    ps = pl.pallas_call(
        lambda *refs: _head_body(shapes, *refs),
        grid=(B,),
        in_specs=in_specs,
        out_specs=out_specs,
        out_shape=out_shape,
        scratch_shapes=scratch_shapes,
        compiler_params=pltpu.CompilerParams(
            dimension_semantics=("arbitrary",)),
    )(*xs, *params)

    outs = []
    for li, (H, W, W_o, W_im) in enumerate(shapes):
        leaves = []
        for j, ch in enumerate((12, 3, 15, 12, 3)):
            t = ps[5 * li + j].reshape(B, 3, ch // 3, H, W_o)
            if W_o != W:
                t = t[..., :W]
            leaves.append(t)
        outs.append(tuple(leaves))
    return tuple(outs)


# speedup vs baseline: 1.0739x; 1.0240x over previous
import jax
import jax.numpy as jnp
from jax.experimental import pallas as pl
from jax.experimental.pallas import tpu as pltpu


def _geom(W):
    W_o = -(-W // 16) * 16
    W_im = -(-(W_o + 2) // 16) * 16
    return W_o, W_im


def _head_body(shapes, *refs):
    n = len(shapes)
    x_refs = refs[:n]
    (wa_ref, ba_ref, w1_ref, s1_ref, b1_ref, w2_ref, s2_ref, b2_ref,
     wp_ref, bp_ref) = refs[n:n + 10]
    o_refs = refs[n + 10:n + 10 + 5 * n]
    h_refs = refs[n + 10 + 5 * n:]

    C = wa_ref.shape[1]
    F = wa_ref.shape[2]
    P = wp_ref.shape[1]

    @pl.when(pl.program_id(0) == 0)
    def _():
        for h in h_refs:
            h[...] = jnp.zeros(h.shape, jnp.bfloat16)

    for i, (H, W, W_o, W_im) in enumerate(shapes):
        x_ref = x_refs[i]
        ob, oo, os_, oa, od = o_refs[5 * i:5 * i + 5]
        h0 = h_refs[2 * i]
        h1 = h_refs[2 * i + 1]
        M = H * W_o

        if W_o != W:
            col = jax.lax.broadcasted_iota(jnp.int32, (M, F), 0) % W_o
            keep = col < W

        def to_im2col(v, dst):
            if W_o != W:
                v = jnp.where(keep, v, 0.0)
            img = v.astype(jnp.bfloat16).reshape(H, W_o, C)
            dst[1:H + 1, 1:W_o + 1, 0:C] = img
            dst[1:H + 1, 0:W_o, C:2 * C] = img
            dst[1:H + 1, 0:W_o - 1, 2 * C:3 * C] = img[:, 1:, :]

        x = x_ref[0].reshape(M, C)
        a = jnp.dot(x, wa_ref[i], preferred_element_type=jnp.float32)
        to_im2col(a + ba_ref[i], h0)

        def conv3(src_ref, w_ref, s_ref, b_ref):
            acc = None
            for di in range(3):
                xs = src_ref[di:di + H, 0:W_o, :].reshape(M, 3 * C)
                d = jnp.dot(xs, w_ref[i, di],
                            preferred_element_type=jnp.float32)
                acc = d if acc is None else acc + d
            return jnp.maximum(acc * s_ref[i] + b_ref[i], 0.0)

        to_im2col(conv3(h0, w1_ref, s1_ref, b1_ref), h1)
        v2 = conv3(h1, w2_ref, s2_ref, b2_ref)

        pT = jax.lax.dot_general(wp_ref[i], v2, (((1,), (1,)), ((), ())),
                                 preferred_element_type=jnp.float32)
        pT = pT + bp_ref[i]
        k = jax.lax.broadcasted_iota(jnp.int32, (P, M), 0) % 15
        pT = jnp.where(k == 4, jax.nn.sigmoid(pT),
                       jnp.where(k == 14, jax.nn.softplus(pT) + 1.0, pT))

        for aidx in range(3):
            base = 15 * aidx
            ob[0, 4 * aidx:4 * aidx + 4] = pT[base:base + 4]
            oo[0, aidx:aidx + 1] = pT[base + 4:base + 5]
            os_[0, 5 * aidx:5 * aidx + 5] = pT[base + 5:base + 10]
            oa[0, 4 * aidx:4 * aidx + 4] = pT[base + 10:base + 14]
            od[0, aidx:aidx + 1] = pT[base + 14:base + 15]


def kernel(feat0, feat1, feat2, feat3, adapt_w, adapt_b, c1_w, c1_b, bn1_g,
           bn1_b, bn1_m, bn1_v, c2_w, c2_b, bn2_g, bn2_b, bn2_m, bn2_v,
           pred_w, pred_b):
    eps = 1e-5
    L, F = adapt_b.shape
    C = feat0.shape[1]
    P = pred_b.shape[1]
    feats = [feat0, feat1, feat2, feat3]
    bf16 = jnp.bfloat16
    B = feat0.shape[0]

    wa = adapt_w.reshape(L, F, C).transpose(0, 2, 1).astype(bf16)
    w1 = c1_w.transpose(0, 3, 4, 2, 1).reshape(L, 3, 3 * C, F).astype(bf16)
    w2 = c2_w.transpose(0, 3, 4, 2, 1).reshape(L, 3, 3 * F, F).astype(bf16)
    wp = pred_w.reshape(L, P, F)

    s1 = (bn1_g / jnp.sqrt(bn1_v + eps)).reshape(L, 1, F)
    b1 = ((c1_b - bn1_m) * s1[:, 0] + bn1_b).reshape(L, 1, F)
    s2 = (bn2_g / jnp.sqrt(bn2_v + eps)).reshape(L, 1, F)
    b2 = ((c2_b - bn2_m) * s2[:, 0] + bn2_b).reshape(L, 1, F)
    ba = adapt_b.reshape(L, 1, F)
    bp = pred_b.reshape(L, P, 1)

    shapes = []
    xs = []
    for f in feats:
        _, _, H, W = f.shape
        W_o, W_im = _geom(W)
        shapes.append((H, W, W_o, W_im))
        x = f.transpose(0, 2, 3, 1)
        if W_o != W:
            x = jnp.pad(x, ((0, 0), (0, 0), (0, W_o - W), (0, 0)))
        xs.append(x.astype(bf16))

    full = lambda a: pl.BlockSpec(a.shape, lambda b: (0,) * a.ndim)
    in_specs = (
        [pl.BlockSpec((1, H, W_o, C), lambda b: (b, 0, 0, 0))
         for (H, W, W_o, W_im) in shapes]
        + [full(a) for a in (wa, ba, w1, s1, b1, w2, s2, b2, wp, bp)]
    )
    out_specs = []
    out_shape = []
    for (H, W, W_o, W_im) in shapes:
        for ch in (12, 3, 15, 12, 3):
            out_specs.append(pl.BlockSpec((1, ch, H * W_o),
                                          lambda b: (b, 0, 0)))
            out_shape.append(
                jax.ShapeDtypeStruct((B, ch, H * W_o), jnp.float32))
    scratch_shapes = []
    for (H, W, W_o, W_im) in shapes:
        scratch_shapes += [pltpu.VMEM((H + 2, W_im, 3 * C), bf16)] * 2

    ps = pl.pallas_call(
        lambda *refs: _head_body(shapes, *refs),
        grid=(B,),
        in_specs=in_specs,
        out_specs=out_specs,
        out_shape=out_shape,
        scratch_shapes=scratch_shapes,
        compiler_params=pltpu.CompilerParams(
            dimension_semantics=("arbitrary",)),
    )(*xs, wa, ba, w1, s1, b1, w2, s2, b2, wp, bp)

    outs = []
    for li, (H, W, W_o, W_im) in enumerate(shapes):
        leaves = []
        for j, ch in enumerate((12, 3, 15, 12, 3)):
            t = ps[5 * li + j].reshape(B, 3, ch // 3, H, W_o)
            if W_o != W:
                t = t[..., :W]
            leaves.append(t)
        outs.append(tuple(leaves))
    return tuple(outs)
